# baseline (speedup 1.0000x reference)
import functools

import jax
import jax.numpy as jnp
from jax import lax
from jax.experimental import pallas as pl
from jax.experimental.pallas import tpu as pltpu
from jax.experimental.pallas import tpu_sc as plsc

BATCH = 16384
EMB_D = 32
ROWS = 1000000
NC = 2
NS = 16
LANES = 16
NW = NC * NS
B_PER_W = BATCH // NW
CHUNK = 128
NCHUNK = B_PER_W // CHUNK
GROUPS = B_PER_W // LANES
VPC = CHUNK // LANES


def _sc_body(uid_hbm, iid_hbm, uemb_hbm, iemb_hbm, ubias_hbm, ibias_hbm,
             gb_hbm, out_hbm, uidx, iidx, dimidx, udat, idat, ub, ib, gbv,
             outv, sem, bsem):
    c = lax.axis_index("c")
    s = lax.axis_index("s")
    wid = s * NC + c

    pltpu.sync_copy(uid_hbm.at[pl.ds(wid * NCHUNK, NCHUNK)], uidx)
    pltpu.sync_copy(iid_hbm.at[pl.ds(wid * NCHUNK, NCHUNK)], iidx)
    pltpu.sync_copy(gb_hbm, gbv)

    bias_handles = []
    for j in range(NCHUNK):
        dst = pl.ds(j * CHUNK, CHUNK)
        bias_handles.append(
            pltpu.async_copy(ubias_hbm.at[uidx.at[j]], ub.at[dst], bsem))
        bias_handles.append(
            pltpu.async_copy(ibias_hbm.at[iidx.at[j]], ib.at[dst], bsem))

    def chunk_body(j, carry):
        handles = []
        for (ids, dat, tab) in ((uidx, udat, uemb_hbm), (iidx, idat, iemb_hbm)):
            for d in range(EMB_D):
                for v in range(VPC):
                    vec = ids[j, pl.ds(v * LANES, LANES)] + (d * ROWS)
                    dimidx[d, pl.ds(v * LANES, LANES)] = vec
            for d in range(EMB_D):
                handles.append(pltpu.async_copy(
                    tab.at[dimidx.at[d]],
                    dat.at[d, pl.ds(j * CHUNK, CHUNK)],
                    sem,
                ))
            for h in handles:
                h.wait()
            handles = []
        return carry

    lax.fori_loop(0, NCHUNK, chunk_body, 0, unroll=False)

    for h in bias_handles:
        h.wait()

    gb = gbv[...]

    def group_body(g, carry):
        base = pl.ds(g * LANES, LANES)
        acc = ub[base] + ib[base] + gb
        for d in range(EMB_D):
            acc = acc + udat[d, base] * idat[d, base]
        outv[base] = acc
        return carry

    lax.fori_loop(0, GROUPS, group_body, 0, unroll=False)

    pltpu.sync_copy(outv, out_hbm.at[pl.ds(wid * B_PER_W, B_PER_W)])


@jax.jit
def _run(uid, iid, uemb, iemb, ubias, ibias, gb16):
    mesh = plsc.VectorSubcoreMesh(core_axis_name="c", subcore_axis_name="s")
    return pl.kernel(
        _sc_body,
        out_type=jax.ShapeDtypeStruct((BATCH,), jnp.float32),
        mesh=mesh,
        scratch_types=[
            pltpu.VMEM((NCHUNK, CHUNK), jnp.int32),
            pltpu.VMEM((NCHUNK, CHUNK), jnp.int32),
            pltpu.VMEM((EMB_D, CHUNK), jnp.int32),
            pltpu.VMEM((EMB_D, B_PER_W), jnp.float32),
            pltpu.VMEM((EMB_D, B_PER_W), jnp.float32),
            pltpu.VMEM((B_PER_W,), jnp.float32),
            pltpu.VMEM((B_PER_W,), jnp.float32),
            pltpu.VMEM((LANES,), jnp.float32),
            pltpu.VMEM((B_PER_W,), jnp.float32),
            pltpu.SemaphoreType.DMA,
            pltpu.SemaphoreType.DMA,
        ],
        compiler_params=pltpu.CompilerParams(
            needs_layout_passes=False, use_tc_tiling_on_sc=False),
    )(uid, iid, uemb, iemb, ubias, ibias, gb16)


def kernel(user_id, item_id, tag_embedding, user_emb_W, item_emb_W,
           user_bias_W, item_bias_W, global_bias):
    del tag_embedding
    uid = user_id.astype(jnp.int32).reshape(BATCH // CHUNK, CHUNK)
    iid = item_id.astype(jnp.int32).reshape(BATCH // CHUNK, CHUNK)
    uemb = user_emb_W.T.reshape(-1)
    iemb = item_emb_W.T.reshape(-1)
    ubias = user_bias_W.reshape(-1)
    ibias = item_bias_W.reshape(-1)
    gb16 = jnp.broadcast_to(global_bias.astype(jnp.float32), (LANES,))
    return _run(uid, iid, uemb, iemb, ubias, ibias, gb16)
